# baseline (speedup 1.0000x reference)
import functools

import jax
import jax.numpy as jnp
from jax import lax
from jax.experimental import pallas as pl
from jax.experimental.pallas import tpu as pltpu
from jax.experimental.pallas import tpu_sc as plsc

N_NODES = 10000
N_EDGES = 160000
D = 256
DH = 128
NS = 16
ROWS_A = 624
ROWS_LAST = N_NODES - (NS - 1) * ROWS_A
ROW0_LAST = (NS - 1) * ROWS_A
EDGES_PER_TILE = N_EDGES // NS
AGG_CHUNK = 80
AGG_DEPTH = 4
GROUP_E = AGG_DEPTH * AGG_CHUNK
N_GROUPS = N_EDGES // GROUP_E
DEG_CHUNK = 2000
BN = 1000

_mesh = plsc.VectorSubcoreMesh(core_axis_name="c", subcore_axis_name="s")
_sc_params = pltpu.CompilerParams(use_tc_tiling_on_sc=False)


@functools.partial(
    pl.kernel,
    out_type=(
        jax.ShapeDtypeStruct((N_NODES, 16), jnp.float32),
        jax.ShapeDtypeStruct((N_NODES, 16), jnp.float32),
    ),
    mesh=_mesh,
    compiler_params=_sc_params,
    scratch_types=[
        pltpu.VMEM((DEG_CHUNK,), jnp.int32),
        pltpu.VMEM((DEG_CHUNK, 16), jnp.float32),
        pltpu.VMEM_SHARED((N_NODES, 16), jnp.float32),
    ],
)
def _degree_kernel(src_hbm, dst_hbm, ones_hbm, zeros_hbm,
                   deg_out_hbm, deg_in_hbm, idx_v, ones_v, shared_deg):
    c = lax.axis_index("c")
    s = lax.axis_index("s")
    row0 = pl.multiple_of(s * ROWS_A, 8)

    @pl.when(s < NS - 1)
    def _():
        pltpu.sync_copy(zeros_hbm.at[pl.ds(0, ROWS_A)],
                        shared_deg.at[pl.ds(row0, ROWS_A)])

    @pl.when(s == NS - 1)
    def _():
        pltpu.sync_copy(zeros_hbm,
                        shared_deg.at[pl.ds(ROW0_LAST, ROWS_LAST)])

    pltpu.sync_copy(ones_hbm, ones_v)
    plsc.subcore_barrier()

    def scatter_ones(ids_hbm):
        def body(j, carry):
            base = pl.multiple_of(s * EDGES_PER_TILE + j * DEG_CHUNK, 16)
            pltpu.sync_copy(ids_hbm.at[pl.ds(base, DEG_CHUNK)], idx_v)
            pltpu.sync_copy(ones_v, shared_deg.at[idx_v], add=True)
            return carry
        lax.fori_loop(0, EDGES_PER_TILE // DEG_CHUNK, body, 0)

    @pl.when(c == 0)
    def _():
        scatter_ones(src_hbm)

    @pl.when(c == 1)
    def _():
        scatter_ones(dst_hbm)

    plsc.subcore_barrier()

    def writeback(out_hbm):
        @pl.when(s < NS - 1)
        def _():
            pltpu.sync_copy(shared_deg.at[pl.ds(row0, ROWS_A)],
                            out_hbm.at[pl.ds(row0, ROWS_A)])

        @pl.when(s == NS - 1)
        def _():
            pltpu.sync_copy(shared_deg.at[pl.ds(ROW0_LAST, ROWS_LAST)],
                            out_hbm.at[pl.ds(ROW0_LAST, ROWS_LAST)])

    @pl.when(c == 0)
    def _():
        writeback(deg_out_hbm)

    @pl.when(c == 1)
    def _():
        writeback(deg_in_hbm)


@functools.partial(
    pl.kernel,
    out_type=(
        jax.ShapeDtypeStruct((N_NODES, DH), jnp.float32),
        jax.ShapeDtypeStruct((N_NODES, DH), jnp.float32),
    ),
    mesh=_mesh,
    compiler_params=_sc_params,
    scratch_types=(
        [pltpu.VMEM((AGG_CHUNK,), jnp.int32)] * AGG_DEPTH
        + [pltpu.VMEM((AGG_CHUNK,), jnp.int32)] * AGG_DEPTH
        + [pltpu.VMEM((AGG_CHUNK, DH), jnp.float32)] * AGG_DEPTH
        + [pltpu.VMEM_SHARED((N_NODES, DH), jnp.float32)]
        + [pltpu.SemaphoreType.DMA] * (2 * AGG_DEPTH)
    ),
)
def _agg_kernel(h0_hbm, h1_hbm, src_hbm, dst_hbm, zeros_hbm,
                agg0_hbm, agg1_hbm, *scratch):
    idx_s = scratch[0:AGG_DEPTH]
    idx_d = scratch[AGG_DEPTH:2 * AGG_DEPTH]
    rows = scratch[2 * AGG_DEPTH:3 * AGG_DEPTH]
    shared_agg = scratch[3 * AGG_DEPTH]
    sg = scratch[3 * AGG_DEPTH + 1:3 * AGG_DEPTH + 1 + AGG_DEPTH]
    ss = scratch[3 * AGG_DEPTH + 1 + AGG_DEPTH:]
    c = lax.axis_index("c")
    s = lax.axis_index("s")
    row0 = pl.multiple_of(s * ROWS_A, 8)

    @pl.when(s < NS - 1)
    def _():
        pltpu.sync_copy(zeros_hbm.at[pl.ds(0, ROWS_A)],
                        shared_agg.at[pl.ds(row0, ROWS_A)])

    @pl.when(s == NS - 1)
    def _():
        pltpu.sync_copy(zeros_hbm,
                        shared_agg.at[pl.ds(ROW0_LAST, ROWS_LAST)])

    plsc.subcore_barrier()

    base_groups = N_GROUPS // NS
    n_extra = N_GROUPS - base_groups * NS
    grp0 = s * base_groups + jnp.minimum(s, n_extra)
    n_groups = base_groups + jnp.where(s < n_extra, 1, 0)

    def run(h_hbm):
        def body(k, carry):
            for m in range(AGG_DEPTH):
                @pl.when(k > 0)
                def _(m=m):
                    pltpu.make_async_copy(
                        rows[m], shared_agg.at[idx_d[m]], ss[m]).wait()
                base = pl.multiple_of(
                    (grp0 + k) * GROUP_E + m * AGG_CHUNK, 16)
                pltpu.sync_copy(src_hbm.at[pl.ds(base, AGG_CHUNK)], idx_s[m])
                pltpu.sync_copy(dst_hbm.at[pl.ds(base, AGG_CHUNK)], idx_d[m])
                pltpu.async_copy(h_hbm.at[idx_s[m]], rows[m], sg[m])
            for m in range(AGG_DEPTH):
                pltpu.make_async_copy(h_hbm.at[idx_s[m]], rows[m],
                                      sg[m]).wait()
                pltpu.async_copy(rows[m], shared_agg.at[idx_d[m]], ss[m],
                                 add=True)
            return carry
        lax.fori_loop(0, n_groups, body, 0)
        for m in range(AGG_DEPTH):
            pltpu.make_async_copy(rows[m], shared_agg.at[idx_d[m]],
                                  ss[m]).wait()

    @pl.when(c == 0)
    def _():
        run(h0_hbm)

    @pl.when(c == 1)
    def _():
        run(h1_hbm)

    plsc.subcore_barrier()

    def writeback(out_hbm):
        @pl.when(s < NS - 1)
        def _():
            pltpu.sync_copy(shared_agg.at[pl.ds(row0, ROWS_A)],
                            out_hbm.at[pl.ds(row0, ROWS_A)])

        @pl.when(s == NS - 1)
        def _():
            pltpu.sync_copy(shared_agg.at[pl.ds(ROW0_LAST, ROWS_LAST)],
                            out_hbm.at[pl.ds(ROW0_LAST, ROWS_LAST)])

    @pl.when(c == 0)
    def _():
        writeback(agg0_hbm)

    @pl.when(c == 1)
    def _():
        writeback(agg1_hbm)


def _scale_split_body(x_ref, deg_ref, o0_ref, o1_ref):
    ns = lax.rsqrt(jnp.maximum(deg_ref[:, 0:1], 1.0))
    xs = x_ref[...] * ns
    o0_ref[...] = xs[:, :DH]
    o1_ref[...] = xs[:, DH:]


def _scale_split(x, deg_out):
    return pl.pallas_call(
        _scale_split_body,
        grid=(N_NODES // BN,),
        in_specs=[
            pl.BlockSpec((BN, D), lambda i: (i, 0)),
            pl.BlockSpec((BN, 16), lambda i: (i, 0)),
        ],
        out_specs=[pl.BlockSpec((BN, DH), lambda i: (i, 0))] * 2,
        out_shape=[jax.ShapeDtypeStruct((N_NODES, DH), jnp.float32)] * 2,
    )(x, deg_out)


def _mid_layer_body(a0_ref, a1_ref, din_ref, dout_ref, W_ref, b_ref,
                    o0_ref, o1_ref):
    nd = lax.rsqrt(jnp.maximum(din_ref[:, 0:1], 1.0))
    h = jnp.concatenate([a0_ref[...], a1_ref[...]], axis=1) * nd
    y = jnp.dot(h, W_ref[...], preferred_element_type=jnp.float32) + b_ref[...]
    y = jnp.maximum(y, 0.0)
    ns = lax.rsqrt(jnp.maximum(dout_ref[:, 0:1], 1.0))
    y = y * ns
    o0_ref[...] = y[:, :DH]
    o1_ref[...] = y[:, DH:]


def _mid_layer(agg0, agg1, deg_in, deg_out, W, b):
    return pl.pallas_call(
        _mid_layer_body,
        grid=(N_NODES // BN,),
        in_specs=[
            pl.BlockSpec((BN, DH), lambda i: (i, 0)),
            pl.BlockSpec((BN, DH), lambda i: (i, 0)),
            pl.BlockSpec((BN, 16), lambda i: (i, 0)),
            pl.BlockSpec((BN, 16), lambda i: (i, 0)),
            pl.BlockSpec((D, D), lambda i: (0, 0)),
            pl.BlockSpec((1, D), lambda i: (0, 0)),
        ],
        out_specs=[pl.BlockSpec((BN, DH), lambda i: (i, 0))] * 2,
        out_shape=[jax.ShapeDtypeStruct((N_NODES, DH), jnp.float32)] * 2,
    )(agg0, agg1, deg_in, deg_out, W, b)


def _final_body(a0_ref, a1_ref, din_ref, W2_ref, b2_ref,
                Wf1_ref, bf1_ref, Wf2_ref, bf2_ref, ans_ref, hg_ref):
    i = pl.program_id(0)
    nd = lax.rsqrt(jnp.maximum(din_ref[:, 0:1], 1.0))
    h = jnp.concatenate([a0_ref[...], a1_ref[...]], axis=1) * nd
    y = jnp.dot(h, W2_ref[...], preferred_element_type=jnp.float32) + b2_ref[...]
    m = jnp.max(y, axis=0, keepdims=True)

    @pl.when(i == 0)
    def _():
        hg_ref[...] = m

    @pl.when(i > 0)
    def _():
        hg_ref[...] = jnp.maximum(hg_ref[...], m)

    @pl.when(i == N_NODES // BN - 1)
    def _():
        hg = hg_ref[...]
        z = jnp.dot(hg, Wf1_ref[...], preferred_element_type=jnp.float32)
        z = jnp.maximum(z + bf1_ref[...], 0.0)
        logit = jnp.dot(z, Wf2_ref[...], preferred_element_type=jnp.float32)
        logit = logit + bf2_ref[...]
        e = jnp.exp(logit - jnp.max(logit, axis=1, keepdims=True))
        ans_ref[...] = e / jnp.sum(e, axis=1, keepdims=True)


def _final(agg0, agg1, deg_in, W2, b2, Wf1, bf1, Wf2, bf2):
    return pl.pallas_call(
        _final_body,
        grid=(N_NODES // BN,),
        in_specs=[
            pl.BlockSpec((BN, DH), lambda i: (i, 0)),
            pl.BlockSpec((BN, DH), lambda i: (i, 0)),
            pl.BlockSpec((BN, 16), lambda i: (i, 0)),
            pl.BlockSpec((D, D), lambda i: (0, 0)),
            pl.BlockSpec((1, D), lambda i: (0, 0)),
            pl.BlockSpec((D, DH), lambda i: (0, 0)),
            pl.BlockSpec((1, DH), lambda i: (0, 0)),
            pl.BlockSpec((DH, 10), lambda i: (0, 0)),
            pl.BlockSpec((1, 10), lambda i: (0, 0)),
        ],
        out_specs=[
            pl.BlockSpec((1, 10), lambda i: (0, 0)),
            pl.BlockSpec((1, D), lambda i: (0, 0)),
        ],
        out_shape=[
            jax.ShapeDtypeStruct((1, 10), jnp.float32),
            jax.ShapeDtypeStruct((1, D), jnp.float32),
        ],
    )(agg0, agg1, deg_in, W2, b2, Wf1, bf1, Wf2, bf2)


def kernel(x, edge_index, W1, b1, W2, b2, Wf1, bf1, Wf2, bf2):
    src = edge_index[0].astype(jnp.int32)
    dst = edge_index[1].astype(jnp.int32)
    ones16 = jnp.ones((DEG_CHUNK, 16), jnp.float32)
    zeros16 = jnp.zeros((ROWS_LAST, 16), jnp.float32)
    zeros128 = jnp.zeros((ROWS_LAST, DH), jnp.float32)

    deg_out, deg_in = _degree_kernel(src, dst, ones16, zeros16)
    xs0, xs1 = _scale_split(x, deg_out)
    agg0, agg1 = _agg_kernel(xs0, xs1, src, dst, zeros128)
    h0, h1 = _mid_layer(agg0, agg1, deg_in, deg_out, W1, b1.reshape(1, D))
    agg0b, agg1b = _agg_kernel(h0, h1, src, dst, zeros128)
    ans, hg = _final(agg0b, agg1b, deg_in, W2, b2.reshape(1, D),
                     Wf1, bf1.reshape(1, DH), Wf2, bf2.reshape(1, 10))
    return (ans, hg)


# speedup vs baseline: 1.1188x; 1.1188x over previous
import functools

import jax
import jax.numpy as jnp
from jax import lax
from jax.experimental import pallas as pl
from jax.experimental.pallas import tpu as pltpu
from jax.experimental.pallas import tpu_sc as plsc

N_NODES = 10000
N_EDGES = 160000
D = 256
DH = 128
NS = 16
ROWS_A = 624
ROWS_LAST = N_NODES - (NS - 1) * ROWS_A
ROW0_LAST = (NS - 1) * ROWS_A
EDGES_PER_TILE = N_EDGES // NS
AGG_CHUNK = 160
PAIR_E = 2 * AGG_CHUNK
N_PAIRS = N_EDGES // PAIR_E
DEG_CHUNK = 2000
BN = 1000

_mesh = plsc.VectorSubcoreMesh(core_axis_name="c", subcore_axis_name="s")
_sc_params = pltpu.CompilerParams(use_tc_tiling_on_sc=False)


@functools.partial(
    pl.kernel,
    out_type=(
        jax.ShapeDtypeStruct((N_NODES, 16), jnp.float32),
        jax.ShapeDtypeStruct((N_NODES, 16), jnp.float32),
    ),
    mesh=_mesh,
    compiler_params=_sc_params,
    scratch_types=[
        pltpu.VMEM((DEG_CHUNK,), jnp.int32),
        pltpu.VMEM((DEG_CHUNK, 16), jnp.float32),
        pltpu.VMEM_SHARED((N_NODES, 16), jnp.float32),
    ],
)
def _degree_kernel(src_hbm, dst_hbm, ones_hbm, zeros_hbm,
                   deg_out_hbm, deg_in_hbm, idx_v, ones_v, shared_deg):
    c = lax.axis_index("c")
    s = lax.axis_index("s")
    row0 = pl.multiple_of(s * ROWS_A, 8)

    @pl.when(s < NS - 1)
    def _():
        pltpu.sync_copy(zeros_hbm.at[pl.ds(0, ROWS_A)],
                        shared_deg.at[pl.ds(row0, ROWS_A)])

    @pl.when(s == NS - 1)
    def _():
        pltpu.sync_copy(zeros_hbm,
                        shared_deg.at[pl.ds(ROW0_LAST, ROWS_LAST)])

    pltpu.sync_copy(ones_hbm, ones_v)
    plsc.subcore_barrier()

    def scatter_ones(ids_hbm):
        def body(j, carry):
            base = pl.multiple_of(s * EDGES_PER_TILE + j * DEG_CHUNK, 16)
            pltpu.sync_copy(ids_hbm.at[pl.ds(base, DEG_CHUNK)], idx_v)
            pltpu.sync_copy(ones_v, shared_deg.at[idx_v], add=True)
            return carry
        lax.fori_loop(0, EDGES_PER_TILE // DEG_CHUNK, body, 0)

    @pl.when(c == 0)
    def _():
        scatter_ones(src_hbm)

    @pl.when(c == 1)
    def _():
        scatter_ones(dst_hbm)

    plsc.subcore_barrier()

    def writeback(out_hbm):
        @pl.when(s < NS - 1)
        def _():
            pltpu.sync_copy(shared_deg.at[pl.ds(row0, ROWS_A)],
                            out_hbm.at[pl.ds(row0, ROWS_A)])

        @pl.when(s == NS - 1)
        def _():
            pltpu.sync_copy(shared_deg.at[pl.ds(ROW0_LAST, ROWS_LAST)],
                            out_hbm.at[pl.ds(ROW0_LAST, ROWS_LAST)])

    @pl.when(c == 0)
    def _():
        writeback(deg_out_hbm)

    @pl.when(c == 1)
    def _():
        writeback(deg_in_hbm)


@functools.partial(
    pl.kernel,
    out_type=(
        jax.ShapeDtypeStruct((N_NODES, DH), jnp.float32),
        jax.ShapeDtypeStruct((N_NODES, DH), jnp.float32),
    ),
    mesh=_mesh,
    compiler_params=_sc_params,
    scratch_types=(
        [pltpu.VMEM((AGG_CHUNK,), jnp.int32)] * 8
        + [pltpu.VMEM((AGG_CHUNK, DH), jnp.float32)] * 2
        + [pltpu.VMEM_SHARED((N_NODES, DH), jnp.float32)]
        + [pltpu.SemaphoreType.DMA] * 12
    ),
)
def _agg_kernel(h0_hbm, h1_hbm, src_hbm, dst_hbm, zeros_hbm,
                agg0_hbm, agg1_hbm, *scratch):
    sa = scratch[0:2]
    sb = scratch[2:4]
    da = scratch[4:6]
    db = scratch[6:8]
    rows0, rows1 = scratch[8:10]
    shared_agg = scratch[10]
    q_sa = scratch[11:13]
    q_sb = scratch[13:15]
    q_da = scratch[15:17]
    q_db = scratch[17:19]
    sg0, sg1, ss0, ss1 = scratch[19:23]
    c = lax.axis_index("c")
    s = lax.axis_index("s")
    row0 = pl.multiple_of(s * ROWS_A, 8)

    @pl.when(s < NS - 1)
    def _():
        pltpu.sync_copy(zeros_hbm.at[pl.ds(0, ROWS_A)],
                        shared_agg.at[pl.ds(row0, ROWS_A)])

    @pl.when(s == NS - 1)
    def _():
        pltpu.sync_copy(zeros_hbm,
                        shared_agg.at[pl.ds(ROW0_LAST, ROWS_LAST)])

    plsc.subcore_barrier()

    base_pairs = N_PAIRS // NS
    n_extra = N_PAIRS - base_pairs * NS
    pair0 = s * base_pairs + jnp.minimum(s, n_extra)
    n_pairs = base_pairs + jnp.where(s < n_extra, 1, 0)
    last_base = pl.multiple_of((pair0 + n_pairs - 1) * PAIR_E, 16)

    def run(h_hbm):
        def idx_issue(m, abase):
            bbase = abase + AGG_CHUNK
            pltpu.async_copy(src_hbm.at[pl.ds(abase, AGG_CHUNK)],
                             sa[m], q_sa[m])
            pltpu.async_copy(src_hbm.at[pl.ds(bbase, AGG_CHUNK)],
                             sb[m], q_sb[m])
            pltpu.async_copy(dst_hbm.at[pl.ds(abase, AGG_CHUNK)],
                             da[m], q_da[m])
            pltpu.async_copy(dst_hbm.at[pl.ds(bbase, AGG_CHUNK)],
                             db[m], q_db[m])

        def idx_wait(m, abase):
            bbase = abase + AGG_CHUNK
            pltpu.make_async_copy(src_hbm.at[pl.ds(abase, AGG_CHUNK)],
                                  sa[m], q_sa[m]).wait()
            pltpu.make_async_copy(src_hbm.at[pl.ds(bbase, AGG_CHUNK)],
                                  sb[m], q_sb[m]).wait()
            pltpu.make_async_copy(dst_hbm.at[pl.ds(abase, AGG_CHUNK)],
                                  da[m], q_da[m]).wait()
            pltpu.make_async_copy(dst_hbm.at[pl.ds(bbase, AGG_CHUNK)],
                                  db[m], q_db[m]).wait()

        def drain(m):
            pltpu.make_async_copy(rows0, shared_agg.at[da[m]], ss0).wait()
            pltpu.make_async_copy(rows1, shared_agg.at[db[m]], ss1).wait()

        def gathers(m):
            pltpu.async_copy(h_hbm.at[sa[m]], rows0, sg0)
            pltpu.async_copy(h_hbm.at[sb[m]], rows1, sg1)

        def scatters(m):
            pltpu.make_async_copy(h_hbm.at[sa[m]], rows0, sg0).wait()
            pltpu.async_copy(rows0, shared_agg.at[da[m]], ss0, add=True)
            pltpu.make_async_copy(h_hbm.at[sb[m]], rows1, sg1).wait()
            pltpu.async_copy(rows1, shared_agg.at[db[m]], ss1, add=True)

        idx_issue(0, pl.multiple_of(pair0 * PAIR_E, 16))

        def body(k, carry):
            abase = pl.multiple_of((pair0 + 2 * k) * PAIR_E, 16)
            idx_wait(0, abase)
            @pl.when(k > 0)
            def _():
                drain(1)
            gathers(0)
            nbase = pl.multiple_of(abase + PAIR_E, 16)
            idx_issue(1, nbase)
            scatters(0)
            idx_wait(1, nbase)
            drain(0)
            gathers(1)
            t2 = jnp.minimum(2 * k + 2, n_pairs - 1) + pair0
            idx_issue(0, pl.multiple_of(t2 * PAIR_E, 16))
            scatters(1)
            return carry
        lax.fori_loop(0, n_pairs // 2, body, 0)

        idx_wait(0, last_base)

        @pl.when(n_pairs % 2 == 1)
        def _():
            drain(1)
            gathers(0)
            scatters(0)
            drain(0)

        @pl.when(n_pairs % 2 == 0)
        def _():
            drain(1)

    @pl.when(c == 0)
    def _():
        run(h0_hbm)

    @pl.when(c == 1)
    def _():
        run(h1_hbm)

    plsc.subcore_barrier()

    def writeback(out_hbm):
        @pl.when(s < NS - 1)
        def _():
            pltpu.sync_copy(shared_agg.at[pl.ds(row0, ROWS_A)],
                            out_hbm.at[pl.ds(row0, ROWS_A)])

        @pl.when(s == NS - 1)
        def _():
            pltpu.sync_copy(shared_agg.at[pl.ds(ROW0_LAST, ROWS_LAST)],
                            out_hbm.at[pl.ds(ROW0_LAST, ROWS_LAST)])

    @pl.when(c == 0)
    def _():
        writeback(agg0_hbm)

    @pl.when(c == 1)
    def _():
        writeback(agg1_hbm)


def _scale_split_body(x_ref, deg_ref, o0_ref, o1_ref):
    ns = lax.rsqrt(jnp.maximum(deg_ref[:, 0:1], 1.0))
    xs = x_ref[...] * ns
    o0_ref[...] = xs[:, :DH]
    o1_ref[...] = xs[:, DH:]


def _scale_split(x, deg_out):
    return pl.pallas_call(
        _scale_split_body,
        grid=(N_NODES // BN,),
        in_specs=[
            pl.BlockSpec((BN, D), lambda i: (i, 0)),
            pl.BlockSpec((BN, 16), lambda i: (i, 0)),
        ],
        out_specs=[pl.BlockSpec((BN, DH), lambda i: (i, 0))] * 2,
        out_shape=[jax.ShapeDtypeStruct((N_NODES, DH), jnp.float32)] * 2,
    )(x, deg_out)


def _mid_layer_body(a0_ref, a1_ref, din_ref, dout_ref, W_ref, b_ref,
                    o0_ref, o1_ref):
    nd = lax.rsqrt(jnp.maximum(din_ref[:, 0:1], 1.0))
    h = jnp.concatenate([a0_ref[...], a1_ref[...]], axis=1) * nd
    y = jnp.dot(h, W_ref[...], preferred_element_type=jnp.float32) + b_ref[...]
    y = jnp.maximum(y, 0.0)
    ns = lax.rsqrt(jnp.maximum(dout_ref[:, 0:1], 1.0))
    y = y * ns
    o0_ref[...] = y[:, :DH]
    o1_ref[...] = y[:, DH:]


def _mid_layer(agg0, agg1, deg_in, deg_out, W, b):
    return pl.pallas_call(
        _mid_layer_body,
        grid=(N_NODES // BN,),
        in_specs=[
            pl.BlockSpec((BN, DH), lambda i: (i, 0)),
            pl.BlockSpec((BN, DH), lambda i: (i, 0)),
            pl.BlockSpec((BN, 16), lambda i: (i, 0)),
            pl.BlockSpec((BN, 16), lambda i: (i, 0)),
            pl.BlockSpec((D, D), lambda i: (0, 0)),
            pl.BlockSpec((1, D), lambda i: (0, 0)),
        ],
        out_specs=[pl.BlockSpec((BN, DH), lambda i: (i, 0))] * 2,
        out_shape=[jax.ShapeDtypeStruct((N_NODES, DH), jnp.float32)] * 2,
    )(agg0, agg1, deg_in, deg_out, W, b)


def _final_body(a0_ref, a1_ref, din_ref, W2_ref, b2_ref,
                Wf1_ref, bf1_ref, Wf2_ref, bf2_ref, ans_ref, hg_ref):
    i = pl.program_id(0)
    nd = lax.rsqrt(jnp.maximum(din_ref[:, 0:1], 1.0))
    h = jnp.concatenate([a0_ref[...], a1_ref[...]], axis=1) * nd
    y = jnp.dot(h, W2_ref[...], preferred_element_type=jnp.float32) + b2_ref[...]
    m = jnp.max(y, axis=0, keepdims=True)

    @pl.when(i == 0)
    def _():
        hg_ref[...] = m

    @pl.when(i > 0)
    def _():
        hg_ref[...] = jnp.maximum(hg_ref[...], m)

    @pl.when(i == N_NODES // BN - 1)
    def _():
        hg = hg_ref[...]
        z = jnp.dot(hg, Wf1_ref[...], preferred_element_type=jnp.float32)
        z = jnp.maximum(z + bf1_ref[...], 0.0)
        logit = jnp.dot(z, Wf2_ref[...], preferred_element_type=jnp.float32)
        logit = logit + bf2_ref[...]
        e = jnp.exp(logit - jnp.max(logit, axis=1, keepdims=True))
        ans_ref[...] = e / jnp.sum(e, axis=1, keepdims=True)


def _final(agg0, agg1, deg_in, W2, b2, Wf1, bf1, Wf2, bf2):
    return pl.pallas_call(
        _final_body,
        grid=(N_NODES // BN,),
        in_specs=[
            pl.BlockSpec((BN, DH), lambda i: (i, 0)),
            pl.BlockSpec((BN, DH), lambda i: (i, 0)),
            pl.BlockSpec((BN, 16), lambda i: (i, 0)),
            pl.BlockSpec((D, D), lambda i: (0, 0)),
            pl.BlockSpec((1, D), lambda i: (0, 0)),
            pl.BlockSpec((D, DH), lambda i: (0, 0)),
            pl.BlockSpec((1, DH), lambda i: (0, 0)),
            pl.BlockSpec((DH, 10), lambda i: (0, 0)),
            pl.BlockSpec((1, 10), lambda i: (0, 0)),
        ],
        out_specs=[
            pl.BlockSpec((1, 10), lambda i: (0, 0)),
            pl.BlockSpec((1, D), lambda i: (0, 0)),
        ],
        out_shape=[
            jax.ShapeDtypeStruct((1, 10), jnp.float32),
            jax.ShapeDtypeStruct((1, D), jnp.float32),
        ],
    )(agg0, agg1, deg_in, W2, b2, Wf1, bf1, Wf2, bf2)


def kernel(x, edge_index, W1, b1, W2, b2, Wf1, bf1, Wf2, bf2):
    src = edge_index[0].astype(jnp.int32)
    dst = edge_index[1].astype(jnp.int32)
    ones16 = jnp.ones((DEG_CHUNK, 16), jnp.float32)
    zeros16 = jnp.zeros((ROWS_LAST, 16), jnp.float32)
    zeros128 = jnp.zeros((ROWS_LAST, DH), jnp.float32)

    deg_out, deg_in = _degree_kernel(src, dst, ones16, zeros16)
    xs0, xs1 = _scale_split(x, deg_out)
    agg0, agg1 = _agg_kernel(xs0, xs1, src, dst, zeros128)
    h0, h1 = _mid_layer(agg0, agg1, deg_in, deg_out, W1, b1.reshape(1, D))
    agg0b, agg1b = _agg_kernel(h0, h1, src, dst, zeros128)
    ans, hg = _final(agg0b, agg1b, deg_in, W2, b2.reshape(1, D),
                     Wf1, bf1.reshape(1, DH), Wf2, bf2.reshape(1, 10))
    return (ans, hg)
